# Initial kernel scaffold; baseline (speedup 1.0000x reference)
import jax
import jax.numpy as jnp
from jax.experimental import pallas as pl


def kernel(x, pred_class, W1, b1, g1, bt1, Wr1, br1, gr1, btr1, Wr2, br2, gr2, btr2, Wf, bf):
    raise NotImplementedError("write your pallas kernel here")


# speedup vs baseline: 5447.4057x; 5447.4057x over previous
import functools

import jax
import jax.numpy as jnp
import numpy as np
from jax.experimental import pallas as pl
from jax.experimental.pallas import tpu as pltpu

_STAGE, _BS, _NQ, _C = 6, 8, 300, 256
_N = _STAGE * _BS * _NQ
_BLK = 512
_NB = 29
_NP = _BLK * _NB
_NPOSE = 216
_EPS = 1e-5
_NEWTON_ITERS = 7

_COF = (
    (4, 8, 5, 7),
    (5, 6, 3, 8),
    (3, 7, 4, 6),
    (2, 7, 1, 8),
    (0, 8, 2, 6),
    (1, 6, 0, 7),
    (1, 5, 2, 4),
    (2, 3, 0, 5),
    (0, 4, 1, 3),
)


def _cof_det(x):
    c = [x[a] * x[b] - x[cc] * x[d] for (a, b, cc, d) in _COF]
    det = x[0] * c[0] + x[1] * c[1] + x[2] * c[2]
    return c, det


def _polar_planes(planes):
    x = list(planes)
    sgn = None
    for k in range(_NEWTON_ITERS):
        c, det = _cof_det(x)
        if k == 0:
            sgn = jnp.where(det < 0.0, -1.0, 1.0)
        dsafe = jnp.where(det >= 0.0,
                          jnp.maximum(det, 1e-30),
                          jnp.minimum(det, -1e-30))
        z = jnp.abs(dsafe) ** (-1.0 / 3.0)
        w = (1.0 / z) / dsafe
        x = [0.5 * (z * xe + w * ce) for xe, ce in zip(x, c)]
    return [xe * sgn for xe in x]


def _body(x_ref, W1_ref, b1_ref, g1_ref, bt1_ref,
          Wr1_ref, br1_ref, gr1_ref, btr1_ref,
          Wr2_ref, br2_ref, gr2_ref, btr2_ref,
          Wfg_ref, bfg_ref, Wbc_ref, bbc_ref, P_ref,
          rot_ref, beta_ref, cam_ref,
          A_scr, B_scr, S_scr):
    p = pl.program_id(0)
    i = pl.program_id(1)
    row0 = i * _BLK
    rows = row0 + jax.lax.broadcasted_iota(jnp.int32, (_BLK, 1), 0)
    wmask = (rows < _N).astype(jnp.float32)

    @pl.when(jnp.logical_and(p == 0, i == 0))
    def _init():
        S_scr[...] = jnp.zeros((8, _C), jnp.float32)

    def acc_stats(k, y):
        ym = y * wmask
        S_scr[k:k + 1, :] += jnp.sum(ym, axis=0, keepdims=True)
        S_scr[k + 1:k + 2, :] += jnp.sum(ym * ym, axis=0, keepdims=True)

    def bn(y, k, g_ref, b_ref):
        mu = S_scr[k:k + 1, :] * (1.0 / _N)
        var = S_scr[k + 1:k + 2, :] * (1.0 / _N) - mu * mu
        return (y - mu) / jnp.sqrt(var + _EPS) * g_ref[...] + b_ref[...]

    @pl.when(p == 0)
    def _p0():
        y = jnp.dot(x_ref[...], W1_ref[...],
                    preferred_element_type=jnp.float32) + b1_ref[...]
        A_scr[pl.ds(row0, _BLK), :] = y
        acc_stats(0, y)

    @pl.when(p == 1)
    def _p1():
        y1 = A_scr[pl.ds(row0, _BLK), :]
        h = jnp.maximum(bn(y1, 0, g1_ref, bt1_ref), 0.0)
        B_scr[pl.ds(row0, _BLK), :] = h
        y2 = jnp.dot(h, Wr1_ref[...],
                     preferred_element_type=jnp.float32) + br1_ref[...]
        A_scr[pl.ds(row0, _BLK), :] = y2
        acc_stats(2, y2)

    @pl.when(p == 2)
    def _p2():
        y2 = A_scr[pl.ds(row0, _BLK), :]
        r = jnp.maximum(bn(y2, 2, gr1_ref, btr1_ref), 0.0)
        y3 = jnp.dot(r, Wr2_ref[...],
                     preferred_element_type=jnp.float32) + br2_ref[...]
        A_scr[pl.ds(row0, _BLK), :] = y3
        acc_stats(4, y3)

    @pl.when(p == 3)
    def _p3():
        y3 = A_scr[pl.ds(row0, _BLK), :]
        h = B_scr[pl.ds(row0, _BLK), :]
        h2 = jnp.maximum(h + bn(y3, 4, gr2_ref, btr2_ref), 0.0)
        bc = jnp.dot(h2, Wbc_ref[...],
                     preferred_element_type=jnp.float32) + bbc_ref[...]
        beta_ref[...] = bc[:, :10]
        cam_ref[...] = bc[:, 10:13]
        rot_g = jnp.dot(h2, Wfg_ref[...],
                        preferred_element_type=jnp.float32) + bfg_ref[...]
        t1 = rot_g.T
        planes = [t1[24 * e:24 * e + 24, :] for e in range(9)]
        q = _polar_planes(planes)
        g = jnp.concatenate(q + [jnp.zeros((40, _BLK), jnp.float32)], axis=0)
        gp = jnp.dot(P_ref[...], g, preferred_element_type=jnp.float32)
        rot_ref[...] = gp.T[:, :_NPOSE]


def _run(xp, W1, b1, g1, bt1, Wr1, br1, gr1, btr1, Wr2, br2, gr2, btr2,
         Wfg, bfg, Wbc, bbc, Pmat):
    cvec = lambda: pl.BlockSpec((1, _C), lambda p, i: (0, 0))
    wmat = lambda n: pl.BlockSpec((_C, n), lambda p, i: (0, 0))
    return pl.pallas_call(
        _body,
        grid=(4, _NB),
        in_specs=[
            pl.BlockSpec((_BLK, _C), lambda p, i: (jnp.where(p == 0, i, 0), 0)),
            wmat(_C), cvec(), cvec(), cvec(),
            wmat(_C), cvec(), cvec(), cvec(),
            wmat(_C), cvec(), cvec(), cvec(),
            wmat(_C), cvec(),
            pl.BlockSpec((_C, 13), lambda p, i: (0, 0)),
            pl.BlockSpec((1, 13), lambda p, i: (0, 0)),
            wmat(_C),
        ],
        out_specs=[
            pl.BlockSpec((_BLK, _NPOSE), lambda p, i: (jnp.where(p == 3, i, _NB), 0)),
            pl.BlockSpec((_BLK, 10), lambda p, i: (jnp.where(p == 3, i, _NB), 0)),
            pl.BlockSpec((_BLK, 3), lambda p, i: (jnp.where(p == 3, i, _NB), 0)),
        ],
        scratch_shapes=[
            pltpu.VMEM((_NP, _C), jnp.float32),
            pltpu.VMEM((_NP, _C), jnp.float32),
            pltpu.VMEM((8, _C), jnp.float32),
        ],
        out_shape=[
            jax.ShapeDtypeStruct((_NP + _BLK, _NPOSE), jnp.float32),
            jax.ShapeDtypeStruct((_NP + _BLK, 10), jnp.float32),
            jax.ShapeDtypeStruct((_NP + _BLK, 3), jnp.float32),
        ],
    )(xp, W1, b1, g1, bt1, Wr1, br1, gr1, btr1, Wr2, br2, gr2, btr2,
      Wfg, bfg, Wbc, bbc, Pmat)


_SRC = np.empty((_NPOSE,), np.int32)
for _g in range(_NPOSE):
    _SRC[_g] = 9 * (_g % 24) + _g // 24
_PM = np.zeros((_C, _C), np.float32)
for _m in range(24):
    for _e in range(9):
        _PM[9 * _m + _e, 24 * _e + _m] = 1.0


def kernel(x, pred_class, W1, b1, g1, bt1, Wr1, br1, gr1, btr1,
           Wr2, br2, gr2, btr2, Wf, bf):
    del pred_class
    xf = x.reshape(-1, _C)
    xp = jnp.pad(xf, ((0, _NP - _N), (0, 0)))
    src = jnp.asarray(_SRC)
    Wfg = jnp.pad(Wf[:, :_NPOSE][:, src], ((0, 0), (0, _C - _NPOSE)))
    bfg = jnp.pad(bf[:_NPOSE][src], (0, _C - _NPOSE)).reshape(1, _C)
    Wbc = Wf[:, _NPOSE:]
    bbc = bf[_NPOSE:].reshape(1, 13)
    r2 = lambda v: v.reshape(1, _C)
    rot, betas, cam = _run(
        xp, W1, r2(b1), r2(g1), r2(bt1),
        Wr1, r2(br1), r2(gr1), r2(btr1),
        Wr2, r2(br2), r2(gr2), r2(btr2),
        Wfg, bfg, Wbc, bbc, jnp.asarray(_PM))
    rotmat = rot[:_N].reshape(_STAGE, _BS, _NQ, 24, 3, 3)
    betas = betas[:_N].reshape(_STAGE, _BS, _NQ, 10)
    camera = cam[:_N].reshape(_STAGE, _BS, _NQ, 3)
    return (rotmat, betas, camera)


# speedup vs baseline: 7320.5667x; 1.3439x over previous
import functools

import jax
import jax.numpy as jnp
import numpy as np
from jax.experimental import pallas as pl
from jax.experimental.pallas import tpu as pltpu

_STAGE, _BS, _NQ, _C = 6, 8, 300, 256
_N = _STAGE * _BS * _NQ
_BLK = 512
_NB = 29
_NP = _BLK * _NB
_NPOSE = 216
_EPS = 1e-5
_NEWTON_ITERS = 7

_COF = (
    (4, 8, 5, 7),
    (5, 6, 3, 8),
    (3, 7, 4, 6),
    (2, 7, 1, 8),
    (0, 8, 2, 6),
    (1, 6, 0, 7),
    (1, 5, 2, 4),
    (2, 3, 0, 5),
    (0, 4, 1, 3),
)


def _cof_det(x):
    c = [x[a] * x[b] - x[cc] * x[d] for (a, b, cc, d) in _COF]
    det = x[0] * c[0] + x[1] * c[1] + x[2] * c[2]
    return c, det


def _polar_planes(planes):
    x = list(planes)
    sgn = None
    for k in range(_NEWTON_ITERS):
        c, det = _cof_det(x)
        if k == 0:
            sgn = jnp.where(det < 0.0, -1.0, 1.0)
        dsafe = jnp.where(det >= 0.0,
                          jnp.maximum(det, 1e-30),
                          jnp.minimum(det, -1e-30))
        z = jnp.abs(dsafe) ** (-1.0 / 3.0)
        w = (1.0 / z) / dsafe
        x = [0.5 * (z * xe + w * ce) for xe, ce in zip(x, c)]
    return [xe * sgn for xe in x]


def _body(x_ref, W1_ref, b1_ref, g1_ref, bt1_ref,
          Wr1_ref, br1_ref, gr1_ref, btr1_ref,
          Wr2_ref, br2_ref, gr2_ref, btr2_ref,
          Wfg_ref, bfg_ref, Wbc_ref, bbc_ref, P_ref,
          rot_ref, beta_ref, cam_ref,
          A_scr, B_scr, S_scr):
    p = pl.program_id(0)
    i = pl.program_id(1)
    row0 = i * _BLK
    rows = row0 + jax.lax.broadcasted_iota(jnp.int32, (_BLK, 1), 0)
    wmask = rows < _N

    @pl.when(jnp.logical_and(p == 0, i == 0))
    def _init():
        S_scr[...] = jnp.zeros((8, _C), jnp.float32)

    def acc_stats(k, y):
        ym = jnp.where(wmask, y, 0.0)
        S_scr[k:k + 1, :] += jnp.sum(ym, axis=0, keepdims=True)
        S_scr[k + 1:k + 2, :] += jnp.sum(ym * ym, axis=0, keepdims=True)

    def bn(y, k, g_ref, b_ref):
        mu = S_scr[k:k + 1, :] * (1.0 / _N)
        var = S_scr[k + 1:k + 2, :] * (1.0 / _N) - mu * mu
        return (y - mu) / jnp.sqrt(var + _EPS) * g_ref[...] + b_ref[...]

    @pl.when(p == 0)
    def _p0():
        y = jnp.dot(x_ref[...], W1_ref[...],
                    preferred_element_type=jnp.float32) + b1_ref[...]
        A_scr[pl.ds(row0, _BLK), :] = y
        acc_stats(0, y)

    @pl.when(p == 1)
    def _p1():
        y1 = A_scr[pl.ds(row0, _BLK), :]
        h = jnp.maximum(bn(y1, 0, g1_ref, bt1_ref), 0.0)
        B_scr[pl.ds(row0, _BLK), :] = h
        y2 = jnp.dot(h, Wr1_ref[...],
                     preferred_element_type=jnp.float32) + br1_ref[...]
        A_scr[pl.ds(row0, _BLK), :] = y2
        acc_stats(2, y2)

    @pl.when(p == 2)
    def _p2():
        y2 = A_scr[pl.ds(row0, _BLK), :]
        r = jnp.maximum(bn(y2, 2, gr1_ref, btr1_ref), 0.0)
        y3 = jnp.dot(r, Wr2_ref[...],
                     preferred_element_type=jnp.float32) + br2_ref[...]
        A_scr[pl.ds(row0, _BLK), :] = y3
        acc_stats(4, y3)

    @pl.when(p == 3)
    def _p3():
        y3 = A_scr[pl.ds(row0, _BLK), :]
        h = B_scr[pl.ds(row0, _BLK), :]
        h2 = jnp.maximum(h + bn(y3, 4, gr2_ref, btr2_ref), 0.0)
        bc = jnp.dot(h2, Wbc_ref[...],
                     preferred_element_type=jnp.float32) + bbc_ref[...]
        beta_ref[...] = bc[:, :10]
        cam_ref[...] = bc[:, 10:13]
        rot_g = jnp.dot(h2, Wfg_ref[...],
                        preferred_element_type=jnp.float32) + bfg_ref[...]
        t1 = rot_g.T
        planes = [t1[24 * e:24 * e + 24, :] for e in range(9)]
        q = _polar_planes(planes)
        g = jnp.concatenate(q + [jnp.zeros((40, _BLK), jnp.float32)], axis=0)
        gp = jnp.dot(P_ref[...], g, preferred_element_type=jnp.float32)
        rot_ref[...] = gp.T[:, :_NPOSE]


def _run(xp, W1, b1, g1, bt1, Wr1, br1, gr1, btr1, Wr2, br2, gr2, btr2,
         Wfg, bfg, Wbc, bbc, Pmat):
    cvec = lambda: pl.BlockSpec((1, _C), lambda p, i: (0, 0))
    wmat = lambda n: pl.BlockSpec((_C, n), lambda p, i: (0, 0))
    return pl.pallas_call(
        _body,
        grid=(4, _NB),
        in_specs=[
            pl.BlockSpec((_BLK, _C), lambda p, i: (jnp.where(p == 0, i, 0), 0)),
            wmat(_C), cvec(), cvec(), cvec(),
            wmat(_C), cvec(), cvec(), cvec(),
            wmat(_C), cvec(), cvec(), cvec(),
            wmat(_C), cvec(),
            pl.BlockSpec((_C, 13), lambda p, i: (0, 0)),
            pl.BlockSpec((1, 13), lambda p, i: (0, 0)),
            wmat(_C),
        ],
        out_specs=[
            pl.BlockSpec((_BLK, _NPOSE), lambda p, i: (jnp.where(p == 3, i, 0), 0)),
            pl.BlockSpec((_BLK, 10), lambda p, i: (jnp.where(p == 3, i, 0), 0)),
            pl.BlockSpec((_BLK, 3), lambda p, i: (jnp.where(p == 3, i, 0), 0)),
        ],
        scratch_shapes=[
            pltpu.VMEM((_NP, _C), jnp.float32),
            pltpu.VMEM((_NP, _C), jnp.float32),
            pltpu.VMEM((8, _C), jnp.float32),
        ],
        out_shape=[
            jax.ShapeDtypeStruct((_N, _NPOSE), jnp.float32),
            jax.ShapeDtypeStruct((_N, 10), jnp.float32),
            jax.ShapeDtypeStruct((_N, 3), jnp.float32),
        ],
    )(xp, W1, b1, g1, bt1, Wr1, br1, gr1, btr1, Wr2, br2, gr2, btr2,
      Wfg, bfg, Wbc, bbc, Pmat)


_SRC = np.empty((_NPOSE,), np.int32)
for _g in range(_NPOSE):
    _SRC[_g] = 9 * (_g % 24) + _g // 24
_PM = np.zeros((_C, _C), np.float32)
for _m in range(24):
    for _e in range(9):
        _PM[9 * _m + _e, 24 * _e + _m] = 1.0


def kernel(x, pred_class, W1, b1, g1, bt1, Wr1, br1, gr1, btr1,
           Wr2, br2, gr2, btr2, Wf, bf):
    del pred_class
    xp = x.reshape(-1, _C)
    src = jnp.asarray(_SRC)
    Wfg = jnp.pad(Wf[:, :_NPOSE][:, src], ((0, 0), (0, _C - _NPOSE)))
    bfg = jnp.pad(bf[:_NPOSE][src], (0, _C - _NPOSE)).reshape(1, _C)
    Wbc = Wf[:, _NPOSE:]
    bbc = bf[_NPOSE:].reshape(1, 13)
    r2 = lambda v: v.reshape(1, _C)
    rot, betas, cam = _run(
        xp, W1, r2(b1), r2(g1), r2(bt1),
        Wr1, r2(br1), r2(gr1), r2(btr1),
        Wr2, r2(br2), r2(gr2), r2(btr2),
        Wfg, bfg, Wbc, bbc, jnp.asarray(_PM))
    rotmat = rot.reshape(_STAGE, _BS, _NQ, 24, 3, 3)
    betas = betas.reshape(_STAGE, _BS, _NQ, 10)
    camera = cam.reshape(_STAGE, _BS, _NQ, 3)
    return (rotmat, betas, camera)


# speedup vs baseline: 8578.0340x; 1.1718x over previous
import jax
import jax.numpy as jnp
import numpy as np
from jax.experimental import pallas as pl
from jax.experimental.pallas import tpu as pltpu

_STAGE, _BS, _NQ, _C = 6, 8, 300, 256
_N = _STAGE * _BS * _NQ
_SEG = _BS * _NQ
_BLK = 512
_NB = 29
_NP = _BLK * _NB
_NPOSE = 216
_EPS = 1e-5
_NEWTON_ITERS = 7

_T0, _T1, _T2, _T3, _T4 = 6, 35, 64, 93, 99

_COF = (
    (4, 8, 5, 7),
    (5, 6, 3, 8),
    (3, 7, 4, 6),
    (2, 7, 1, 8),
    (0, 8, 2, 6),
    (1, 6, 0, 7),
    (1, 5, 2, 4),
    (2, 3, 0, 5),
    (0, 4, 1, 3),
)


def _cof_det(x):
    c = [x[a] * x[b] - x[cc] * x[d] for (a, b, cc, d) in _COF]
    det = x[0] * c[0] + x[1] * c[1] + x[2] * c[2]
    return c, det


def _polar_planes(planes):
    x = list(planes)
    sgn = None
    for k in range(_NEWTON_ITERS):
        c, det = _cof_det(x)
        if k == 0:
            sgn = jnp.where(det < 0.0, -1.0, 1.0)
        dsafe = jnp.where(det >= 0.0,
                          jnp.maximum(det, 1e-30),
                          jnp.minimum(det, -1e-30))
        z = jnp.abs(dsafe) ** (-1.0 / 3.0)
        w = (1.0 / z) / dsafe
        x = [0.5 * (z * xe + w * ce) for xe, ce in zip(x, c)]
    return [xe * sgn for xe in x]


def _body(x_ref, W1_ref, b1_ref, g1_ref, bt1_ref,
          Wr1_ref, br1_ref, gr1_ref, btr1_ref,
          Wr2_ref, br2_ref, gr2_ref, btr2_ref,
          Wfg_ref, bfg_ref, Wbc_ref, bbc_ref, P_ref,
          rot_ref, beta_ref, cam_ref,
          A_scr, B_scr, S_scr):
    t = pl.program_id(0)

    @pl.when(t == 0)
    def _init():
        S_scr[...] = jnp.zeros((8, _C), jnp.float32)

    def acc_stats(k, y, valid=None):
        ym = y if valid is None else jnp.where(valid, y, 0.0)
        S_scr[k:k + 1, :] += jnp.sum(ym, axis=0, keepdims=True)
        S_scr[k + 1:k + 2, :] += jnp.sum(ym * ym, axis=0, keepdims=True)

    def bn(y, k, g_ref, b_ref):
        mu = S_scr[k:k + 1, :] * (1.0 / _N)
        var = S_scr[k + 1:k + 2, :] * (1.0 / _N) - mu * mu
        return (y - mu) / jnp.sqrt(var + _EPS) * g_ref[...] + b_ref[...]

    @pl.when(t < _T0)
    def _p0():
        xv = x_ref[0]
        xcat = jnp.concatenate([xv[j] for j in range(_BS)], axis=0)
        y = jnp.dot(xcat, W1_ref[...],
                    preferred_element_type=jnp.float32) + b1_ref[...]
        A_scr[pl.ds(t * _SEG, _SEG), :] = y
        acc_stats(0, y)

    @pl.when(jnp.logical_and(t >= _T0, t < _T1))
    def _p1():
        i = t - _T0
        row0 = i * _BLK
        rows = row0 + jax.lax.broadcasted_iota(jnp.int32, (_BLK, 1), 0)
        valid = rows < _N
        y1 = A_scr[pl.ds(row0, _BLK), :]
        h = jnp.maximum(bn(y1, 0, g1_ref, bt1_ref), 0.0)
        B_scr[pl.ds(row0, _BLK), :] = h
        y2 = jnp.dot(h, Wr1_ref[...],
                     preferred_element_type=jnp.float32) + br1_ref[...]
        A_scr[pl.ds(row0, _BLK), :] = y2
        acc_stats(2, y2, valid)

    @pl.when(jnp.logical_and(t >= _T1, t < _T2))
    def _p2():
        i = t - _T1
        row0 = i * _BLK
        rows = row0 + jax.lax.broadcasted_iota(jnp.int32, (_BLK, 1), 0)
        valid = rows < _N
        y2 = A_scr[pl.ds(row0, _BLK), :]
        r = jnp.maximum(bn(y2, 2, gr1_ref, btr1_ref), 0.0)
        y3 = jnp.dot(r, Wr2_ref[...],
                     preferred_element_type=jnp.float32) + br2_ref[...]
        A_scr[pl.ds(row0, _BLK), :] = y3
        acc_stats(4, y3, valid)

    @pl.when(jnp.logical_and(t >= _T2, t < _T3))
    def _p3():
        i = t - _T2
        row0 = i * _BLK
        y3 = A_scr[pl.ds(row0, _BLK), :]
        h = B_scr[pl.ds(row0, _BLK), :]
        h2 = jnp.maximum(h + bn(y3, 4, gr2_ref, btr2_ref), 0.0)
        bc = jnp.dot(h2, Wbc_ref[...],
                     preferred_element_type=jnp.float32) + bbc_ref[...]
        rot_g = jnp.dot(h2, Wfg_ref[...],
                        preferred_element_type=jnp.float32) + bfg_ref[...]
        t1 = rot_g.T
        planes = [t1[24 * e:24 * e + 24, :] for e in range(9)]
        q = _polar_planes(planes)
        g = jnp.concatenate(q + [jnp.zeros((40, _BLK), jnp.float32)], axis=0)
        gp = jnp.dot(P_ref[...], g, preferred_element_type=jnp.float32)
        B_scr[pl.ds(row0, _BLK), 0:_NPOSE] = gp.T[:, :_NPOSE]
        B_scr[pl.ds(row0, _BLK), _NPOSE:_NPOSE + 13] = bc

    @pl.when(t >= _T3)
    def _p4():
        u = t - _T3
        sv = B_scr[pl.ds(u * _SEG, _SEG), :]
        parts_r = [sv[300 * j:300 * (j + 1), 0:_NPOSE] for j in range(_BS)]
        parts_b = [sv[300 * j:300 * (j + 1), _NPOSE:_NPOSE + 10]
                   for j in range(_BS)]
        parts_c = [sv[300 * j:300 * (j + 1), _NPOSE + 10:_NPOSE + 13]
                   for j in range(_BS)]
        rot_ref[...] = jnp.stack(parts_r, axis=0)[None]
        beta_ref[...] = jnp.stack(parts_b, axis=0)[None]
        cam_ref[...] = jnp.stack(parts_c, axis=0)[None]


def _run(x4, W1, b1, g1, bt1, Wr1, br1, gr1, btr1, Wr2, br2, gr2, btr2,
         Wfg, bfg, Wbc, bbc, Pmat):
    cvec = lambda: pl.BlockSpec((1, _C), lambda t: (0, 0))
    wmat = lambda n: pl.BlockSpec((_C, n), lambda t: (0, 0))
    oidx = lambda t: (jnp.clip(t - _T3, 0, _STAGE - 1), 0, 0, 0)
    return pl.pallas_call(
        _body,
        grid=(_T4,),
        in_specs=[
            pl.BlockSpec((1, _BS, _NQ, _C),
                         lambda t: (jnp.minimum(t, _STAGE - 1), 0, 0, 0)),
            wmat(_C), cvec(), cvec(), cvec(),
            wmat(_C), cvec(), cvec(), cvec(),
            wmat(_C), cvec(), cvec(), cvec(),
            wmat(_C), cvec(),
            pl.BlockSpec((_C, 13), lambda t: (0, 0)),
            pl.BlockSpec((1, 13), lambda t: (0, 0)),
            wmat(_C),
        ],
        out_specs=[
            pl.BlockSpec((1, _BS, _NQ, _NPOSE), oidx),
            pl.BlockSpec((1, _BS, _NQ, 10), oidx),
            pl.BlockSpec((1, _BS, _NQ, 3), oidx),
        ],
        scratch_shapes=[
            pltpu.VMEM((_NP, _C), jnp.float32),
            pltpu.VMEM((_NP, _C), jnp.float32),
            pltpu.VMEM((8, _C), jnp.float32),
        ],
        out_shape=[
            jax.ShapeDtypeStruct((_STAGE, _BS, _NQ, _NPOSE), jnp.float32),
            jax.ShapeDtypeStruct((_STAGE, _BS, _NQ, 10), jnp.float32),
            jax.ShapeDtypeStruct((_STAGE, _BS, _NQ, 3), jnp.float32),
        ],
    )(x4, W1, b1, g1, bt1, Wr1, br1, gr1, btr1, Wr2, br2, gr2, btr2,
      Wfg, bfg, Wbc, bbc, Pmat)


_SRC = np.empty((_NPOSE,), np.int32)
for _g in range(_NPOSE):
    _SRC[_g] = 9 * (_g % 24) + _g // 24
_PM = np.zeros((_C, _C), np.float32)
for _m in range(24):
    for _e in range(9):
        _PM[9 * _m + _e, 24 * _e + _m] = 1.0


def kernel(x, pred_class, W1, b1, g1, bt1, Wr1, br1, gr1, btr1,
           Wr2, br2, gr2, btr2, Wf, bf):
    del pred_class
    src = jnp.asarray(_SRC)
    Wfg = jnp.pad(Wf[:, :_NPOSE][:, src], ((0, 0), (0, _C - _NPOSE)))
    bfg = jnp.pad(bf[:_NPOSE][src], (0, _C - _NPOSE)).reshape(1, _C)
    Wbc = Wf[:, _NPOSE:]
    bbc = bf[_NPOSE:].reshape(1, 13)
    r2 = lambda v: v.reshape(1, _C)
    rot, betas, cam = _run(
        x, W1, r2(b1), r2(g1), r2(bt1),
        Wr1, r2(br1), r2(gr1), r2(btr1),
        Wr2, r2(br2), r2(gr2), r2(btr2),
        Wfg, bfg, Wbc, bbc, jnp.asarray(_PM))
    rotmat = rot.reshape(_STAGE, _BS, _NQ, 24, 3, 3)
    return (rotmat, betas, cam)
